# baseline (speedup 1.0000x reference)
import jax
import jax.numpy as jnp
from jax.experimental import pallas as pl
from jax.experimental.pallas import tpu as pltpu

EMBED = 128
RING = 14
LOOKAHEAD = 10
BUF_ROWS = 8000

_USER_CHUNKS = [1000, 2000, 5000] + [8000] * 11 + [4000]
_ITEM_CHUNKS = [8000] * 6 + [2000]


def _work_list():
    work = []
    off = 0
    for sz in _USER_CHUNKS:
        work.append((0, off, sz))
        off += sz
    off = 0
    for sz in _ITEM_CHUNKS:
        work.append((1, off, sz))
        off += sz
    return work


def _dma_copy(u_in, i_in, u_out, i_out, *rest):
    bufs = rest[:RING]
    rsem, wsem = rest[RING], rest[RING + 1]
    srcs = (u_in, i_in)
    dsts = (u_out, i_out)
    work = _work_list()
    n = len(work)

    reads = [None] * n
    writes = [None] * n

    def start_read(j):
        t, off, sz = work[j]
        b = j % RING
        rd = pltpu.make_async_copy(
            srcs[t].at[pl.ds(off, sz)], bufs[b].at[pl.ds(0, sz)], rsem.at[b])
        rd.start()
        reads[j] = rd

    for j in range(LOOKAHEAD):
        start_read(j)
    for i in range(n):
        j = i + LOOKAHEAD
        if j < n:
            if j >= RING:
                writes[j - RING].wait()
            start_read(j)
        reads[i].wait()
        t, off, sz = work[i]
        b = i % RING
        wr = pltpu.make_async_copy(
            bufs[b].at[pl.ds(0, sz)], dsts[t].at[pl.ds(off, sz)], wsem.at[b])
        wr.start()
        writes[i] = wr
    for i in range(max(0, n - RING), n):
        writes[i].wait()


def kernel(embed_user, embed_item):
    return tuple(pl.pallas_call(
        _dma_copy,
        in_specs=[
            pl.BlockSpec(memory_space=pl.ANY),
            pl.BlockSpec(memory_space=pl.ANY),
        ],
        out_specs=[
            pl.BlockSpec(memory_space=pl.ANY),
            pl.BlockSpec(memory_space=pl.ANY),
        ],
        out_shape=[
            jax.ShapeDtypeStruct(embed_user.shape, embed_user.dtype),
            jax.ShapeDtypeStruct(embed_item.shape, embed_item.dtype),
        ],
        scratch_shapes=(
            [pltpu.VMEM((BUF_ROWS, EMBED), jnp.float32)] * RING
            + [pltpu.SemaphoreType.DMA((RING,)),
               pltpu.SemaphoreType.DMA((RING,))]
        ),
    )(embed_user, embed_item))


# speedup vs baseline: 1.0115x; 1.0115x over previous
import jax
import jax.numpy as jnp
from jax.experimental import pallas as pl

N_GRID = 5
USER_ROWS = 20000
ITEM_ROWS = 10000
EMBED = 128


def _copy_kernel(user_in, item_in, user_out, item_out):
    user_out[...] = user_in[...]
    item_out[...] = item_in[...]


def kernel(embed_user, embed_item):
    return tuple(pl.pallas_call(
        _copy_kernel,
        grid=(N_GRID,),
        in_specs=[
            pl.BlockSpec((USER_ROWS, EMBED), lambda i: (i, 0)),
            pl.BlockSpec((ITEM_ROWS, EMBED), lambda i: (i, 0)),
        ],
        out_specs=[
            pl.BlockSpec((USER_ROWS, EMBED), lambda i: (i, 0)),
            pl.BlockSpec((ITEM_ROWS, EMBED), lambda i: (i, 0)),
        ],
        out_shape=[
            jax.ShapeDtypeStruct(embed_user.shape, embed_user.dtype),
            jax.ShapeDtypeStruct(embed_item.shape, embed_item.dtype),
        ],
    )(embed_user, embed_item))
